# Initial kernel scaffold; baseline (speedup 1.0000x reference)
import jax
import jax.numpy as jnp
from jax.experimental import pallas as pl


def kernel(input_tensor, many_slices):
    raise NotImplementedError("write your pallas kernel here")


# speedup vs baseline: 4.6475x; 4.6475x over previous
import functools

import jax
import jax.numpy as jnp
from jax import lax
from jax.experimental import pallas as pl
from jax.experimental.pallas import tpu as pltpu
from jax.experimental.pallas import tpu_sc as plsc

B = 4096
F = 100
E = 32
G = 26
S = 4
N = G * B * S
NW = 32
ROWS_W = N // NW
SB_ROWS = 1664
NSB = ROWS_W // SB_ROWS
DMA_ROWS = 128
NDMA = SB_ROWS // DMA_ROWS


def _sc_gather(x3, ms_flat):
    mesh = plsc.VectorSubcoreMesh(core_axis_name="c", subcore_axis_name="s")

    @functools.partial(
        pl.kernel,
        out_type=jax.ShapeDtypeStruct((N, E), jnp.float32),
        mesh=mesh,
        scratch_types=[
            pltpu.VMEM((2 * G * S,), jnp.int32),
            pltpu.VMEM((SB_ROWS,), jnp.int32),
            pltpu.VMEM((SB_ROWS, E), jnp.float32),
            pltpu.SemaphoreType.DMA,
        ],
        compiler_params=pltpu.CompilerParams(
            use_tc_tiling_on_sc=False, needs_layout_passes=False),
    )
    def k(x_hbm, ms_hbm, out_hbm, ms_v, idx_v, data_v, sem):
        wid = lax.axis_index("s") * 2 + lax.axis_index("c")
        row0 = wid * ROWS_W
        pltpu.sync_copy(ms_hbm, ms_v)
        lanes = lax.iota(jnp.int32, 16)
        four = jnp.full((16,), S, jnp.int32)
        bvec = jnp.full((16,), B, jnp.int32)
        evec = jnp.full((16,), E, jnp.int32)
        s_lane = lax.rem(lanes, four)
        q_lane = lax.div(lanes, four)

        for sb in range(NSB):
            sb_base = row0 + sb * SB_ROWS

            def compute_idx(kk, _):
                i0 = sb_base + kk * 16
                q = jnp.full((16,), lax.div(i0, S), jnp.int32) + q_lane
                b = lax.rem(q, bvec)
                g = lax.div(q, bvec)
                start = plsc.load_gather(ms_v, [g * (2 * S) + s_lane * 2])
                idx_v[pl.ds(kk * 16, 16)] = b * F + lax.div(start, evec)
                return 0

            lax.fori_loop(0, SB_ROWS // 16, compute_idx, 0)

            copies = []
            for j in range(NDMA):
                copies.append(pltpu.async_copy(
                    x_hbm.at[idx_v.at[pl.ds(j * DMA_ROWS, DMA_ROWS)]],
                    data_v.at[pl.ds(j * DMA_ROWS, DMA_ROWS)],
                    sem,
                ))
            for c in copies:
                c.wait()
            pltpu.sync_copy(data_v, out_hbm.at[pl.ds(sb_base, SB_ROWS)])

    return k(x3, ms_flat)


def kernel(input_tensor, many_slices):
    x3 = input_tensor.reshape(B * F, E)
    ms_flat = jnp.asarray(many_slices).astype(jnp.int32).reshape(-1)
    out = _sc_gather(x3, ms_flat)
    return out.reshape(G, B, S * E)


# speedup vs baseline: 4.9037x; 1.0551x over previous
import functools

import jax
import jax.numpy as jnp
from jax import lax
from jax.experimental import pallas as pl
from jax.experimental.pallas import tpu as pltpu
from jax.experimental.pallas import tpu_sc as plsc

B = 4096
F = 100
E = 32
G = 26
S = 4
N = G * B * S
NW = 32
ROWS_W = N // NW
SB_ROWS = 1664
NSB = ROWS_W // SB_ROWS
DMA_ROWS = 128
NDMA = SB_ROWS // DMA_ROWS


def _sc_gather(x3, ms_flat):
    mesh = plsc.VectorSubcoreMesh(core_axis_name="c", subcore_axis_name="s")

    @functools.partial(
        pl.kernel,
        out_type=jax.ShapeDtypeStruct((N, E), jnp.float32),
        mesh=mesh,
        scratch_types=[
            pltpu.VMEM((2 * G * S,), jnp.int32),
            pltpu.VMEM((SB_ROWS,), jnp.int32),
            pltpu.VMEM((SB_ROWS,), jnp.int32),
            pltpu.VMEM((SB_ROWS, E), jnp.float32),
            pltpu.VMEM((SB_ROWS, E), jnp.float32),
            pltpu.SemaphoreType.DMA,
            pltpu.SemaphoreType.DMA,
            pltpu.SemaphoreType.DMA,
            pltpu.SemaphoreType.DMA,
        ],
        compiler_params=pltpu.CompilerParams(
            use_tc_tiling_on_sc=False, needs_layout_passes=False),
    )
    def k(x_hbm, ms_hbm, out_hbm, ms_v, idx0, idx1, data0, data1,
          gsem0, gsem1, wsem0, wsem1):
        idx = [idx0, idx1]
        data = [data0, data1]
        gsem = [gsem0, gsem1]
        wsem = [wsem0, wsem1]
        wid = lax.axis_index("s") * 2 + lax.axis_index("c")
        row0 = wid * ROWS_W
        pltpu.sync_copy(ms_hbm, ms_v)
        lanes = lax.iota(jnp.int32, 16)
        four = jnp.full((16,), S, jnp.int32)
        bvec = jnp.full((16,), B, jnp.int32)
        evec = jnp.full((16,), E, jnp.int32)
        s_lane = lax.rem(lanes, four)
        q_lane = lax.div(lanes, four)

        def compute_idx(sb, buf):
            sb_base = row0 + sb * SB_ROWS

            def body(kk, _):
                i0 = sb_base + kk * 16
                q = jnp.full((16,), lax.div(i0, S), jnp.int32) + q_lane
                b = lax.rem(q, bvec)
                g = lax.div(q, bvec)
                start = plsc.load_gather(ms_v, [g * (2 * S) + s_lane * 2])
                idx[buf][pl.ds(kk * 16, 16)] = b * F + lax.div(start, evec)
                return 0

            lax.fori_loop(0, SB_ROWS // 16, body, 0)

        def issue_gathers(buf):
            return [
                pltpu.async_copy(
                    x_hbm.at[idx[buf].at[pl.ds(j * DMA_ROWS, DMA_ROWS)]],
                    data[buf].at[pl.ds(j * DMA_ROWS, DMA_ROWS)],
                    gsem[buf],
                )
                for j in range(NDMA)
            ]

        compute_idx(0, 0)
        gathers = issue_gathers(0)
        writes = [None, None]
        for sb in range(NSB):
            cur, nxt = sb % 2, (sb + 1) % 2
            if sb + 1 < NSB:
                compute_idx(sb + 1, nxt)
            for c in gathers:
                c.wait()
            if writes[cur] is not None:
                writes[cur].wait()
            writes[cur] = pltpu.async_copy(
                data[cur], out_hbm.at[pl.ds(row0 + sb * SB_ROWS, SB_ROWS)],
                wsem[cur])
            if sb + 1 < NSB:
                if writes[nxt] is not None:
                    writes[nxt].wait()
                    writes[nxt] = None
                gathers = issue_gathers(nxt)
        for w in writes:
            if w is not None:
                w.wait()

    return k(x3, ms_flat)


def kernel(input_tensor, many_slices):
    x3 = input_tensor.reshape(B * F, E)
    ms_flat = jnp.asarray(many_slices).astype(jnp.int32).reshape(-1)
    out = _sc_gather(x3, ms_flat)
    return out.reshape(G, B, S * E)


# speedup vs baseline: 4.9582x; 1.0111x over previous
import functools

import jax
import jax.numpy as jnp
from jax import lax
from jax.experimental import pallas as pl
from jax.experimental.pallas import tpu as pltpu
from jax.experimental.pallas import tpu_sc as plsc

B = 4096
F = 100
E = 32
G = 26
S = 4
N = G * B * S
NW = 32
ROWS_W = N // NW
SB_ROWS = 1664
NSB = ROWS_W // SB_ROWS
DMA_ROWS = 128
NDMA = SB_ROWS // DMA_ROWS


def _sc_gather(x3, ms_flat):
    mesh = plsc.VectorSubcoreMesh(core_axis_name="c", subcore_axis_name="s")

    @functools.partial(
        pl.kernel,
        out_type=jax.ShapeDtypeStruct((N, E), jnp.float32),
        mesh=mesh,
        scratch_types=[
            pltpu.VMEM((2 * G * S,), jnp.int32),
            pltpu.VMEM((SB_ROWS,), jnp.int32),
            pltpu.VMEM((SB_ROWS,), jnp.int32),
            pltpu.VMEM((SB_ROWS, E), jnp.float32),
            pltpu.VMEM((SB_ROWS, E), jnp.float32),
            pltpu.SemaphoreType.DMA,
            pltpu.SemaphoreType.DMA,
            pltpu.SemaphoreType.DMA,
            pltpu.SemaphoreType.DMA,
        ],
        compiler_params=pltpu.CompilerParams(
            use_tc_tiling_on_sc=False, needs_layout_passes=False),
    )
    def k(x_hbm, ms_hbm, out_hbm, ms_v, idx0, idx1, data0, data1,
          gsem0, gsem1, wsem0, wsem1):
        idx = [idx0, idx1]
        data = [data0, data1]
        gsem = [gsem0, gsem1]
        wsem = [wsem0, wsem1]
        wid = lax.axis_index("s") * 2 + lax.axis_index("c")
        row0 = wid * ROWS_W
        pltpu.sync_copy(ms_hbm, ms_v)
        lanes = lax.iota(jnp.int32, 16)
        four = jnp.full((16,), S, jnp.int32)
        bvec = jnp.full((16,), B, jnp.int32)
        evec = jnp.full((16,), E, jnp.int32)
        s_lane = lax.rem(lanes, four)
        q_lane = lax.div(lanes, four)

        def compute_idx(sb, buf):
            sb_base = row0 + sb * SB_ROWS

            def body(kk, _):
                i0 = sb_base + kk * 16
                q = jnp.full((16,), lax.div(i0, S), jnp.int32) + q_lane
                b = lax.rem(q, bvec)
                g = lax.div(q, bvec)
                start = plsc.load_gather(ms_v, [g * (2 * S) + s_lane * 2])
                idx[buf][pl.ds(kk * 16, 16)] = b * F + lax.div(start, evec)
                return 0

            lax.fori_loop(0, SB_ROWS // 16, body, 0)

        def issue_gathers(buf):
            return [pltpu.async_copy(x_hbm.at[idx[buf]], data[buf], gsem[buf])]

        compute_idx(0, 0)
        gathers = issue_gathers(0)
        writes = [None, None]
        for sb in range(NSB):
            cur, nxt = sb % 2, (sb + 1) % 2
            if sb + 1 < NSB:
                compute_idx(sb + 1, nxt)
            for c in gathers:
                c.wait()
            if writes[cur] is not None:
                writes[cur].wait()
            writes[cur] = pltpu.async_copy(
                data[cur], out_hbm.at[pl.ds(row0 + sb * SB_ROWS, SB_ROWS)],
                wsem[cur])
            if sb + 1 < NSB:
                if writes[nxt] is not None:
                    writes[nxt].wait()
                    writes[nxt] = None
                gathers = issue_gathers(nxt)
        for w in writes:
            if w is not None:
                w.wait()

    return k(x3, ms_flat)


def kernel(input_tensor, many_slices):
    x3 = input_tensor.reshape(B * F, E)
    ms_flat = jnp.asarray(many_slices).astype(jnp.int32).reshape(-1)
    out = _sc_gather(x3, ms_flat)
    return out.reshape(G, B, S * E)


# speedup vs baseline: 4.9932x; 1.0071x over previous
import functools

import jax
import jax.numpy as jnp
from jax import lax
from jax.experimental import pallas as pl
from jax.experimental.pallas import tpu as pltpu
from jax.experimental.pallas import tpu_sc as plsc

B = 4096
F = 100
E = 32
G = 26
S = 4
N = G * B * S
NW = 32
ROWS_W = N // NW
SB_ROWS = 1024
NSB = ROWS_W // SB_ROWS
NBUF = 3


def _sc_gather(x3, ms_flat):
    mesh = plsc.VectorSubcoreMesh(core_axis_name="c", subcore_axis_name="s")

    @functools.partial(
        pl.kernel,
        out_type=jax.ShapeDtypeStruct((N, E), jnp.float32),
        mesh=mesh,
        scratch_types=(
            [pltpu.VMEM((2 * G * S,), jnp.int32)]
            + [pltpu.VMEM((SB_ROWS,), jnp.int32)] * NBUF
            + [pltpu.VMEM((SB_ROWS, E), jnp.float32)] * NBUF
            + [pltpu.SemaphoreType.DMA] * (2 * NBUF)
        ),
        compiler_params=pltpu.CompilerParams(
            use_tc_tiling_on_sc=False, needs_layout_passes=False),
    )
    def k(x_hbm, ms_hbm, out_hbm, ms_v, *bufs):
        idx = list(bufs[0:NBUF])
        data = list(bufs[NBUF:2 * NBUF])
        gsem = list(bufs[2 * NBUF:3 * NBUF])
        wsem = list(bufs[3 * NBUF:4 * NBUF])
        wid = lax.axis_index("s") * 2 + lax.axis_index("c")
        row0 = wid * ROWS_W
        pltpu.sync_copy(ms_hbm, ms_v)
        lanes = lax.iota(jnp.int32, 16)
        four = jnp.full((16,), S, jnp.int32)
        bvec = jnp.full((16,), B, jnp.int32)
        evec = jnp.full((16,), E, jnp.int32)
        s_lane = lax.rem(lanes, four)
        q_lane = lax.div(lanes, four)

        def compute_idx(sb, buf):
            sb_base = row0 + sb * SB_ROWS

            def body(kk, _):
                i0 = sb_base + kk * 16
                q = jnp.full((16,), lax.div(i0, S), jnp.int32) + q_lane
                b = lax.rem(q, bvec)
                g = lax.div(q, bvec)
                start = plsc.load_gather(ms_v, [g * (2 * S) + s_lane * 2])
                idx[buf][pl.ds(kk * 16, 16)] = b * F + lax.div(start, evec)
                return 0

            lax.fori_loop(0, SB_ROWS // 16, body, 0)

        def issue_gather(buf):
            return pltpu.async_copy(x_hbm.at[idx[buf]], data[buf], gsem[buf])

        def issue_write(sb, buf):
            return pltpu.async_copy(
                data[buf], out_hbm.at[pl.ds(row0 + sb * SB_ROWS, SB_ROWS)],
                wsem[buf])

        gd = [None] * NSB
        wd = [None] * NSB
        for sb in range(NSB):
            buf = sb % NBUF
            if sb >= NBUF:
                wd[sb - NBUF].wait()
            compute_idx(sb, buf)
            gd[sb] = issue_gather(buf)
            if sb >= 1:
                gd[sb - 1].wait()
                wd[sb - 1] = issue_write(sb - 1, (sb - 1) % NBUF)
        gd[NSB - 1].wait()
        wd[NSB - 1] = issue_write(NSB - 1, (NSB - 1) % NBUF)
        for sb in range(NSB - NBUF + 1, NSB):
            wd[sb].wait()

    return k(x3, ms_flat)


def kernel(input_tensor, many_slices):
    x3 = input_tensor.reshape(B * F, E)
    ms_flat = jnp.asarray(many_slices).astype(jnp.int32).reshape(-1)
    out = _sc_gather(x3, ms_flat)
    return out.reshape(G, B, S * E)
